# Initial kernel scaffold; baseline (speedup 1.0000x reference)
import jax
import jax.numpy as jnp
from jax.experimental import pallas as pl


def kernel(x, emb, W1, b1, W2, b2):
    raise NotImplementedError("write your pallas kernel here")


# speedup vs baseline: 7.4955x; 7.4955x over previous
import functools

import jax
import jax.numpy as jnp
from jax import lax
from jax.experimental import pallas as pl
from jax.experimental.pallas import tpu as pltpu, tpu_sc as plsc

VOCAB = 10000
EMBED = 16
SEQ = 20
HID = 16

_info = plsc.get_sparse_core_info()
_NC, _NS, _LANES = _info.num_cores, _info.num_subcores, _info.num_lanes
_NW = _NC * _NS


def _t2_body(emb_ref, w1_ref, out_ref):
    out_ref[0] = jnp.dot(emb_ref[...], w1_ref[0],
                         preferred_element_type=jnp.float32)


def _build_t2(emb, w1_3d):
    return pl.pallas_call(
        _t2_body,
        grid=(SEQ,),
        in_specs=[
            pl.BlockSpec((VOCAB, EMBED), lambda l: (0, 0)),
            pl.BlockSpec((1, EMBED, HID), lambda l: (l, 0, 0)),
        ],
        out_specs=pl.BlockSpec((1, VOCAB, HID), lambda l: (l, 0, 0)),
        out_shape=jax.ShapeDtypeStruct((SEQ, VOCAB, HID), jnp.float32),
    )(emb, w1_3d)


def _make_sc_kernel(batch):
    rows_w = batch // _NW
    idx_w = rows_w * SEQ
    rows_c = 64
    idx_c = rows_c * SEQ
    g_per_c = idx_c // 128
    n_chunks = rows_w // rows_c
    n_groups = rows_c // 16

    mesh = plsc.VectorSubcoreMesh(core_axis_name="c", subcore_axis_name="s")

    @functools.partial(
        pl.kernel,
        mesh=mesh,
        compiler_params=pltpu.CompilerParams(use_tc_tiling_on_sc=False),
        out_type=jax.ShapeDtypeStruct((batch, HID), jnp.float32),
        scratch_types=[
            pltpu.VMEM((idx_w,), jnp.int32),
            pltpu.VMEM((idx_w,), jnp.int32),
            pltpu.VMEM((idx_c, HID), jnp.float32),
            pltpu.VMEM((rows_w, HID), jnp.float32),
            pltpu.SemaphoreType.DMA,
        ],
    )
    def sc_kernel(xf_hbm, t2_hbm, out_hbm, raw_v, idx_v, buf_v, o_v, sem):
        wid = lax.axis_index("s") * _NC + lax.axis_index("c")
        row0 = wid * rows_w

        pltpu.sync_copy(xf_hbm.at[pl.ds(row0 * SEQ, idx_w)], raw_v)
        lane = lax.iota(jnp.int32, 16)
        zero = jnp.zeros((16,), jnp.float32)

        def prep(i, _):
            pos = lane + i * 16
            idx_v[pl.ds(i * 16, 16)] = (
                raw_v[pl.ds(i * 16, 16)] + (pos % SEQ) * VOCAB)
            return 0
        lax.fori_loop(0, idx_w // 16, prep, 0)

        def chunk(c, _):
            copies = []
            for j in range(g_per_c):
                copies.append(pltpu.async_copy(
                    t2_hbm.at[idx_v.at[pl.ds((c * g_per_c + j) * 128, 128)]],
                    buf_v.at[pl.ds(j * 128, 128)],
                    sem))
            for cp in copies:
                cp.wait()

            def group(g, _):
                for rr in range(16):
                    acc = zero
                    for l in range(SEQ):
                        acc = acc + buf_v[g * (16 * SEQ) + rr * SEQ + l, :]
                    o_v[c * rows_c + g * 16 + rr, :] = acc
                return 0
            lax.fori_loop(0, n_groups, group, 0)
            return 0
        lax.fori_loop(0, n_chunks, chunk, 0)

        pltpu.sync_copy(o_v, out_hbm.at[pl.ds(row0, rows_w)])

    return sc_kernel


def _head_body(acc_ref, b1_ref, w2_ref, b2_ref, out_ref):
    h = jnp.maximum(acc_ref[...] + b1_ref[...], 0.0)
    z = jnp.dot(h, w2_ref[...], preferred_element_type=jnp.float32)
    out_ref[...] = 1.0 / (1.0 + jnp.exp(-(z + b2_ref[...])))


def _apply_head(acc, b1, w2, b2):
    batch = acc.shape[0]
    blk = 2048
    return pl.pallas_call(
        _head_body,
        grid=(batch // blk,),
        in_specs=[
            pl.BlockSpec((blk, HID), lambda i: (i, 0)),
            pl.BlockSpec((1, HID), lambda i: (0, 0)),
            pl.BlockSpec((HID, 1), lambda i: (0, 0)),
            pl.BlockSpec((1, 1), lambda i: (0, 0)),
        ],
        out_specs=pl.BlockSpec((blk, 1), lambda i: (i, 0)),
        out_shape=jax.ShapeDtypeStruct((batch, 1), jnp.float32),
    )(acc, b1, w2, b2)


def kernel(x, emb, W1, b1, W2, b2):
    batch = x.shape[0]
    t2 = _build_t2(emb.astype(jnp.float32),
                   W1.astype(jnp.float32).reshape(SEQ, EMBED, HID))
    t2f = t2.reshape(SEQ * VOCAB, HID)
    xf = x.astype(jnp.int32).reshape(-1)
    acc = _make_sc_kernel(batch)(xf, t2f)
    return _apply_head(acc,
                       b1.astype(jnp.float32).reshape(1, HID),
                       W2.astype(jnp.float32),
                       b2.astype(jnp.float32).reshape(1, 1))
